# baseline (speedup 1.0000x reference)
import functools

import jax
import jax.numpy as jnp
from jax import lax
from jax.experimental import pallas as pl
from jax.experimental.pallas import tpu as pltpu
from jax.experimental.pallas import tpu_sc as plsc

_NC = 2
_NS = 16
_NW = _NC * _NS
_L = 16
_NB = 2
_TB = 2


@functools.partial(jax.jit, static_argnums=())
def _sc_pred(e_weight, hid2, chf):
    N, D = hid2.shape
    C = chf.shape[0] // N
    tpw = N // _NW
    mesh = plsc.VectorSubcoreMesh(core_axis_name="c", subcore_axis_name="s")

    @functools.partial(
        pl.kernel,
        out_type=jax.ShapeDtypeStruct((N, C), jnp.float32),
        mesh=mesh,
        scratch_types=[
            pltpu.VMEM((tpw * C,), jnp.int32),
            pltpu.VMEM((tpw, D), jnp.float32),
            pltpu.VMEM((_NB, _TB * C, D), jnp.float32),
            pltpu.VMEM((tpw, C), jnp.float32),
        ] + [pltpu.SemaphoreType.DMA] * _NB,
        compiler_params=pltpu.CompilerParams(
            use_tc_tiling_on_sc=False, needs_layout_passes=False),
    )
    def k(table_hbm, hid_hbm, ch_hbm, out_hbm, ch_v, hid_v, rows_v, pred_v,
          *sems):
        wid = lax.axis_index("s") * _NC + lax.axis_index("c")
        base = wid * tpw
        pltpu.sync_copy(ch_hbm.at[pl.ds(base * C, tpw * C)], ch_v)
        pltpu.sync_copy(hid_hbm.at[pl.ds(base, tpw)], hid_v)

        lane = lax.iota(jnp.int32, _L)
        _A = 4
        _NK = D // _L
        zeros = jnp.zeros((_L,), jnp.float32)

        def _rne(v):
            u = plsc.bitcast(v, jnp.uint32)
            u = ((u + jnp.uint32(0x7FFF) + ((u >> 16) & jnp.uint32(1)))
                 & jnp.uint32(0xFFFF0000))
            return plsc.bitcast(u, jnp.float32)

        def hid_rne_body(t, carry):
            for kk in range(_NK):
                sl = pl.ds(kk * _L, _L)
                hid_v[t, sl] = _rne(hid_v[t, sl])
            return carry

        lax.fori_loop(0, tpw, hid_rne_body, 0, unroll=False)

        nsteps = tpw // _TB
        nrows = _TB * C

        for b in range(_NB):
            pltpu.async_copy(
                table_hbm.at[ch_v.at[pl.ds(b * nrows, nrows)]],
                rows_v.at[b], sems[b])

        def compute_tok(t, b, tt):
            def cand_body(c, carry):
                vec0, vec1 = carry
                accs = [zeros] * _A
                for kk in range(_NK):
                    w = _rne(rows_v[b, tt * C + c, pl.ds(kk * _L, _L)])
                    h = hid_v[t, pl.ds(kk * _L, _L)]
                    accs[kk % _A] = accs[kk % _A] + w * h
                tot = jnp.sum((accs[0] + accs[1]) + (accs[2] + accs[3]))
                hit = lane == (c & (_L - 1))
                vec0 = jnp.where(hit & (c < _L), tot, vec0)
                vec1 = jnp.where(hit & (c >= _L), tot, vec1)
                return vec0, vec1

            vec0, vec1 = lax.fori_loop(0, C, cand_body, (zeros, zeros),
                                       unroll=False)
            pred_v[t, pl.ds(0, _L)] = vec0
            pred_v[t, pl.ds(_L, _L)] = vec1

        def ring_body(p, carry):
            for b in range(_NB):
                step = p * _NB + b
                pltpu.make_async_copy(
                    table_hbm.at[pl.ds(0, nrows)], rows_v.at[b],
                    sems[b]).wait()
                for tt in range(_TB):
                    compute_tok(step * _TB + tt, b, tt)

                @pl.when(step + _NB < nsteps)
                def _():
                    pltpu.async_copy(
                        table_hbm.at[
                            ch_v.at[pl.ds((step + _NB) * nrows, nrows)]],
                        rows_v.at[b], sems[b])
            return carry

        lax.fori_loop(0, nsteps // _NB, ring_body, 0, unroll=False)
        pltpu.sync_copy(pred_v, out_hbm.at[pl.ds(base, tpw)])

    return k(e_weight, hid2, chf)


def _tail(pred, ch2, tags2):
    N, C = pred.shape

    def body(p_ref, c_ref, t_ref, loss_ref, y_ref):
        p = p_ref[...]
        ch = c_ref[...]
        tg = t_ref[...]
        iota = lax.broadcasted_iota(jnp.int32, (N, C), 1)
        m = jnp.max(p, axis=1, keepdims=True)
        am = jnp.min(jnp.where(p == m, iota, C), axis=1, keepdims=True)
        y_ref[...] = jnp.sum(jnp.where(iota == am, ch, 0), axis=1, keepdims=True)
        tval = jnp.sum(jnp.where(iota == tg, p, 0.0), axis=1, keepdims=True)
        s = jnp.sum(jnp.exp(p - m), axis=1, keepdims=True)
        nll = m + jnp.log(s) - tval
        loss_ref[...] = jnp.broadcast_to(jnp.sum(nll) / N, (1, 1))

    return pl.pallas_call(
        body,
        out_shape=(
            jax.ShapeDtypeStruct((1, 1), jnp.float32),
            jax.ShapeDtypeStruct((N, 1), jnp.int32),
        ),
    )(pred, ch2, tags2)


def kernel(hid, choices, tags, e_weight):
    B, T, D = hid.shape
    C = choices.shape[-1]
    N = B * T
    hid2 = hid.reshape(N, D)
    ch2 = choices.reshape(N, C)
    pred = _sc_pred(e_weight, hid2, ch2.reshape(N * C))
    loss, y = _tail(pred, ch2, tags.reshape(N, 1))
    return loss[0, 0], y.reshape(B, T)


# speedup vs baseline: 1.0240x; 1.0240x over previous
import functools

import jax
import jax.numpy as jnp
from jax import lax
from jax.experimental import pallas as pl
from jax.experimental.pallas import tpu as pltpu
from jax.experimental.pallas import tpu_sc as plsc

_NC = 2
_NS = 16
_NW = _NC * _NS
_L = 16
_NB = 2
_TB = 2


@functools.partial(jax.jit, static_argnums=())
def _sc_pred(e_weight, hid2, chf):
    N, D = hid2.shape
    C = chf.shape[0] // N
    tpw = N // _NW
    mesh = plsc.VectorSubcoreMesh(core_axis_name="c", subcore_axis_name="s")

    @functools.partial(
        pl.kernel,
        out_type=jax.ShapeDtypeStruct((N, C), jnp.float32),
        mesh=mesh,
        scratch_types=[
            pltpu.VMEM((tpw * C,), jnp.int32),
            pltpu.VMEM((tpw, D), jnp.float32),
            pltpu.VMEM((_NB, _TB * C, D), jnp.float32),
            pltpu.VMEM((tpw, C), jnp.float32),
        ] + [pltpu.SemaphoreType.DMA] * _NB,
        compiler_params=pltpu.CompilerParams(
            use_tc_tiling_on_sc=False, needs_layout_passes=False),
    )
    def k(table_hbm, hid_hbm, ch_hbm, out_hbm, ch_v, hid_v, rows_v, pred_v,
          *sems):
        wid = lax.axis_index("s") * _NC + lax.axis_index("c")
        base = wid * tpw
        pltpu.sync_copy(ch_hbm.at[pl.ds(base * C, tpw * C)], ch_v)
        pltpu.sync_copy(hid_hbm.at[pl.ds(base, tpw)], hid_v)

        lane = lax.iota(jnp.int32, _L)
        _A = 4
        _NK = D // _L
        zeros = jnp.zeros((_L,), jnp.float32)

        def _rne(v):
            u = plsc.bitcast(v, jnp.uint32)
            u = ((u + jnp.uint32(0x7FFF) + ((u >> 16) & jnp.uint32(1)))
                 & jnp.uint32(0xFFFF0000))
            return plsc.bitcast(u, jnp.float32)

        def hid_rne_body(t, carry):
            for kk in range(_NK):
                sl = pl.ds(kk * _L, _L)
                hid_v[t, sl] = _rne(hid_v[t, sl])
            return carry

        lax.fori_loop(0, tpw, hid_rne_body, 0, unroll=False)

        nsteps = tpw // _TB
        nrows = _TB * C

        for b in range(_NB):
            pltpu.async_copy(
                table_hbm.at[ch_v.at[pl.ds(b * nrows, nrows)]],
                rows_v.at[b], sems[b])

        def compute_tok(t, b, tt):
            def cand_body(c, carry):
                vec0, vec1 = carry
                accs = [zeros] * _A
                for kk in range(_NK):
                    w = _rne(rows_v[b, tt * C + c, pl.ds(kk * _L, _L)])
                    h = hid_v[t, pl.ds(kk * _L, _L)]
                    accs[kk % _A] = accs[kk % _A] + w * h
                tot = jnp.sum((accs[0] + accs[1]) + (accs[2] + accs[3]))
                hit = lane == (c & (_L - 1))
                vec0 = jnp.where(hit & (c < _L), tot, vec0)
                vec1 = jnp.where(hit & (c >= _L), tot, vec1)
                return vec0, vec1

            vec0, vec1 = lax.fori_loop(0, C, cand_body, (zeros, zeros),
                                       unroll=4)
            pred_v[t, pl.ds(0, _L)] = vec0
            pred_v[t, pl.ds(_L, _L)] = vec1

        def ring_body(p, carry):
            for b in range(_NB):
                step = p * _NB + b
                pltpu.make_async_copy(
                    table_hbm.at[pl.ds(0, nrows)], rows_v.at[b],
                    sems[b]).wait()
                for tt in range(_TB):
                    compute_tok(step * _TB + tt, b, tt)

                @pl.when(step + _NB < nsteps)
                def _():
                    pltpu.async_copy(
                        table_hbm.at[
                            ch_v.at[pl.ds((step + _NB) * nrows, nrows)]],
                        rows_v.at[b], sems[b])
            return carry

        lax.fori_loop(0, nsteps // _NB, ring_body, 0, unroll=False)
        pltpu.sync_copy(pred_v, out_hbm.at[pl.ds(base, tpw)])

    return k(e_weight, hid2, chf)


def _tail(pred, ch2, tags2):
    N, C = pred.shape

    def body(p_ref, c_ref, t_ref, loss_ref, y_ref):
        p = p_ref[...]
        ch = c_ref[...]
        tg = t_ref[...]
        iota = lax.broadcasted_iota(jnp.int32, (N, C), 1)
        m = jnp.max(p, axis=1, keepdims=True)
        am = jnp.min(jnp.where(p == m, iota, C), axis=1, keepdims=True)
        y_ref[...] = jnp.sum(jnp.where(iota == am, ch, 0), axis=1, keepdims=True)
        tval = jnp.sum(jnp.where(iota == tg, p, 0.0), axis=1, keepdims=True)
        s = jnp.sum(jnp.exp(p - m), axis=1, keepdims=True)
        nll = m + jnp.log(s) - tval
        loss_ref[...] = jnp.broadcast_to(jnp.sum(nll) / N, (1, 1))

    return pl.pallas_call(
        body,
        out_shape=(
            jax.ShapeDtypeStruct((1, 1), jnp.float32),
            jax.ShapeDtypeStruct((N, 1), jnp.int32),
        ),
    )(pred, ch2, tags2)


def kernel(hid, choices, tags, e_weight):
    B, T, D = hid.shape
    C = choices.shape[-1]
    N = B * T
    hid2 = hid.reshape(N, D)
    ch2 = choices.reshape(N, C)
    pred = _sc_pred(e_weight, hid2, ch2.reshape(N * C))
    loss, y = _tail(pred, ch2, tags.reshape(N, 1))
    return loss[0, 0], y.reshape(B, T)
